# baseline (speedup 1.0000x reference)
import jax
import jax.numpy as jnp
from jax import lax
from jax.experimental import pallas as pl
from jax.experimental.pallas import tpu as pltpu
from jax.experimental.pallas import tpu_sc as plsc

F32 = jnp.float32
I32 = jnp.int32

N, E, D, H, CC = 10000, 320000, 128, 4, 32
NN = 10240
NC, NS, NW = 2, 16, 32
EPW = E // NW
S = 80
NCHUNK = EPW // S
NPAIR = (NCHUNK - 1) // 2
NPT = NN // NS

_SCPARAMS = dict(compiler_params=pltpu.CompilerParams(needs_layout_passes=False))


def _splat(v):
  return jnp.broadcast_to(v, (16,)).astype(I32)


def _a1_body(x_ref, wl_ref, bl_ref, wr_ref, br_ref, xl_ref, xr_ref):
  xb = x_ref[...]
  hp = jax.lax.Precision.HIGHEST
  xl_ref[...] = jnp.dot(xb, wl_ref[...], preferred_element_type=F32,
                        precision=hp) + bl_ref[...]
  xr_ref[...] = jnp.dot(xb, wr_ref[...], preferred_element_type=F32,
                        precision=hp) + br_ref[...]


def _a2_body(ea_ref, wee_ref, bee_ref, we_ref, out_ref):
  hp = jax.lax.Precision.HIGHEST
  t = jnp.dot(ea_ref[...], wee_ref[...], preferred_element_type=F32,
              precision=hp) + bee_ref[...]
  out_ref[...] = jnp.dot(t, we_ref[...], preferred_element_type=F32,
                         precision=hp)


def _c1_body(sa0_ref, sa1_ref, d0_ref, d1_ref, wee_ref, bee_ref, we_ref,
             le_ref):
  hp = jax.lax.Precision.HIGHEST
  dsum = d0_ref[...] + d1_ref[...]
  inv = 1.0 / jnp.maximum(dsum, 1.0)
  mean = (sa0_ref[...] + sa1_ref[...]) * inv
  t = jnp.dot(mean, wee_ref[...], preferred_element_type=F32,
              precision=hp) + bee_ref[...]
  t = jnp.where(dsum > 0.0, t, 0.0)
  le_ref[...] = jnp.dot(t, we_ref[...], preferred_element_type=F32,
                        precision=hp)


def _c2_body(d0_ref, d1_ref, inv_ref):
  d = d0_ref[...] + d1_ref[...] + 1.0
  inv_ref[...] = 1.0 / (d + 1e-16)


def _f_body(o0_ref, o1_ref, inv_ref, xl_ref, bias_ref, wlin_ref,
            blin_ref, out_ref):
  blk = o0_ref.shape[0]
  inv = inv_ref[...]
  cols = [jnp.broadcast_to(inv[:, h:h + 1], (blk, CC)) for h in range(H)]
  al = jnp.concatenate(cols, axis=1)
  acc = o0_ref[...] + o1_ref[...] + xl_ref[...] * al + bias_ref[...]
  acc = jnp.maximum(acc, 0.0)
  out_ref[...] = jnp.dot(acc, wlin_ref[...], preferred_element_type=F32,
                         precision=jax.lax.Precision.HIGHEST) + blin_ref[...]


def _b0_body(dst_h, ea_h, sega_h, deg_h,
             dst0_v, dst1_v, ea0_v, ea1_v, ones_v, dbuf_v,
             sega_acc, deg_acc, semA, semB):
  c = lax.axis_index("c")
  s = lax.axis_index("s")
  w = c * NS + s
  z16 = jnp.zeros((16,), F32)
  o16 = jnp.ones((16,), F32)

  def fill_ones(i, _):
    ones_v[pl.ds(i * 16, 16)] = o16
    return 0
  lax.fori_loop(0, S // 16, fill_ones, 0)

  def fill_zb(i, _):
    ea0_v[i // 8, pl.ds((i % 8) * 16, 16)] = z16
    return 0
  lax.fori_loop(0, S * 8, fill_zb, 0)

  def fill_db(i, _):
    dbuf_v[pl.ds(i * 16, 16)] = z16
    return 0
  lax.fori_loop(0, NPT // 16, fill_db, 0)

  def zcp(k, _):
    pltpu.sync_copy(ea0_v, sega_acc.at[pl.ds(s * NPT + k * S, S)])
    return 0
  lax.fori_loop(0, NPT // S, zcp, 0)
  pltpu.sync_copy(dbuf_v, deg_acc.at[pl.ds(s * NPT, NPT)])
  plsc.subcore_barrier()

  def pair(j, _):
    baseA = w * EPW + (2 * j) * S
    baseB = baseA + S
    cpA = pltpu.async_copy(ea_h.at[pl.ds(baseA, S)], ea0_v, semA)
    cpB = pltpu.async_copy(ea_h.at[pl.ds(baseB, S)], ea1_v, semB)
    pltpu.sync_copy(dst_h.at[pl.ds(baseA, S)], dst0_v)
    pltpu.sync_copy(dst_h.at[pl.ds(baseB, S)], dst1_v)
    cpA.wait()
    pltpu.sync_copy(ea0_v, sega_acc.at[dst0_v], add=True)
    pltpu.sync_copy(ones_v, deg_acc.at[dst0_v], add=True)
    cpB.wait()
    pltpu.sync_copy(ea1_v, sega_acc.at[dst1_v], add=True)
    pltpu.sync_copy(ones_v, deg_acc.at[dst1_v], add=True)
    return 0
  lax.fori_loop(0, NPAIR, pair, 0)

  baseZ = w * EPW + (NCHUNK - 1) * S
  cpZ = pltpu.async_copy(ea_h.at[pl.ds(baseZ, S)], ea0_v, semA)
  pltpu.sync_copy(dst_h.at[pl.ds(baseZ, S)], dst0_v)
  cpZ.wait()
  pltpu.sync_copy(ea0_v, sega_acc.at[dst0_v], add=True)
  pltpu.sync_copy(ones_v, deg_acc.at[dst0_v], add=True)

  plsc.subcore_barrier()
  def bounce(k, _):
    r0 = s * NPT + k * S
    pltpu.sync_copy(sega_acc.at[pl.ds(r0, S)], ea0_v)
    pltpu.sync_copy(ea0_v, sega_h.at[c, pl.ds(r0, S)])
    return 0
  lax.fori_loop(0, NPT // S, bounce, 0)
  pltpu.sync_copy(deg_acc.at[pl.ds(s * NPT, NPT)], dbuf_v)
  pltpu.sync_copy(dbuf_v, deg_h.at[c, pl.ds(s * NPT, NPT)])


def _b1_body(src_h, dst_h, xl_h, xr_h, ee_h, att_h,
             logits_h,
             src0_v, dst0_v, src1_v, dst1_v,
             xl0_v, xr0_v, ee0_v, xl1_v, xr1_v, ee1_v,
             lg_v, lg2_v, att_v, semA, semB, semC):
  c = lax.axis_index("c")
  s = lax.axis_index("s")
  w = c * NS + s
  iota = lax.iota(I32, 16)
  z16 = jnp.zeros((16,), F32)

  pltpu.sync_copy(att_h, att_v)

  def issue(base, srcv, dstv, xlv, xrv, eev, sem):
    pltpu.sync_copy(src_h.at[pl.ds(base, S)], srcv)
    pltpu.sync_copy(dst_h.at[pl.ds(base, S)], dstv)
    return [
        pltpu.async_copy(xl_h.at[srcv], xlv, sem),
        pltpu.async_copy(xr_h.at[dstv], xrv, sem),
        pltpu.async_copy(ee_h.at[pl.ds(base, S)], eev, sem),
    ]

  def compute(base, xlv, xrv, eev, lgv, sem):
    for g in range(S // 16):
      ridx = iota + g * 16
      def cbody(cc, carry):
        dcol = jnp.bitwise_and(iota + cc, CC - 1)
        accs = []
        for h in range(H):
          ch = dcol + h * CC
          a = plsc.load_gather(xlv, [ridx, ch])
          b = plsc.load_gather(xrv, [ridx, ch])
          e = plsc.load_gather(eev, [ridx, ch])
          av = plsc.load_gather(att_v, [ch])
          f = a + b + e
          f = jnp.maximum(f, 0.2 * f)
          accs.append(carry[h] + f * av)
        return tuple(accs)
      accs = plsc.parallel_loop(0, CC, 1, unroll=4,
                                carry=(z16, z16, z16, z16))(cbody)
      for h in range(H):
        plsc.store_scatter(lgv, [ridx * H + h], accs[h])
    pltpu.sync_copy(lgv, logits_h.at[pl.ds(base * H, S * H)])

  def pair(j, _):
    baseA = w * EPW + (2 * j) * S
    baseB = baseA + S
    cpsA = issue(baseA, src0_v, dst0_v, xl0_v, xr0_v, ee0_v, semA)
    cpsB = issue(baseB, src1_v, dst1_v, xl1_v, xr1_v, ee1_v, semB)
    for cp in cpsA:
      cp.wait()
    compute(baseA, xl0_v, xr0_v, ee0_v, lg_v, semC)
    for cp in cpsB:
      cp.wait()
    compute(baseB, xl1_v, xr1_v, ee1_v, lg2_v, semC)
    return 0
  lax.fori_loop(0, NPAIR, pair, 0)

  baseZ = w * EPW + (NCHUNK - 1) * S
  cpsZ = issue(baseZ, src0_v, dst0_v, xl0_v, xr0_v, ee0_v, semA)
  for cp in cpsZ:
    cp.wait()
  compute(baseZ, xl0_v, xr0_v, ee0_v, lg_v, semC)


def _d_body(dst_h, xl_h, xr_h, le_h, att_h, logits_h,
            expv_h, den_h,
            dst_v, lg_v, ev_v, dst2_v, lg2_v, ev2_v,
            xl_v, xr_v, le_v, mbuf_v, db_v, att_v, m_vmem,
            eh0_v, eh1_v, eh2_v, eh3_v, ih0_v, ih1_v, ih2_v, ih3_v,
            eh4_v, eh5_v, eh6_v, eh7_v, ih4_v, ih5_v, ih6_v, ih7_v,
            m_spmem, den_acc, sem1, sem2, sem3):
  c = lax.axis_index("c")
  s = lax.axis_index("s")
  w = c * NS + s
  iota = lax.iota(I32, 16)
  z16 = jnp.zeros((16,), F32)
  eh = [eh0_v, eh1_v, eh2_v, eh3_v]
  ih = [ih0_v, ih1_v, ih2_v, ih3_v]
  eh2 = [eh4_v, eh5_v, eh6_v, eh7_v]
  ih2 = [ih4_v, ih5_v, ih6_v, ih7_v]

  pltpu.sync_copy(att_h, att_v)

  def fill_db(i, _):
    db_v[pl.ds(i * 16, 16)] = z16
    return 0
  lax.fori_loop(0, (NPT * H) // 16, fill_db, 0)
  pltpu.sync_copy(db_v, den_acc.at[pl.ds(s * NPT * H, NPT * H)])

  def sub(k, _):
    n0 = s * NPT + k * 64
    cp1 = pltpu.async_copy(xl_h.at[pl.ds(n0, 64)], xl_v, sem1)
    cp2 = pltpu.async_copy(xr_h.at[pl.ds(n0, 64)], xr_v, sem2)
    cp3 = pltpu.async_copy(le_h.at[pl.ds(n0, 64)], le_v, sem3)
    cp1.wait()
    cp2.wait()
    cp3.wait()
    for g in range(4):
      ridx = iota + g * 16
      def cbody(cc, carry):
        dcol = jnp.bitwise_and(iota + cc, CC - 1)
        accs = []
        for h in range(H):
          ch = dcol + h * CC
          a = plsc.load_gather(xl_v, [ridx, ch])
          b = plsc.load_gather(xr_v, [ridx, ch])
          e = plsc.load_gather(le_v, [ridx, ch])
          f = a + b + e
          f = jnp.maximum(f, 0.2 * f)
          av = plsc.load_gather(att_v, [ch])
          accs.append(carry[h] + f * av)
        return tuple(accs)
      accs = plsc.parallel_loop(0, CC, 1, unroll=4,
                                carry=(z16, z16, z16, z16))(cbody)
      for h in range(H):
        plsc.store_scatter(mbuf_v, [ridx * H + h], accs[h])
    pltpu.sync_copy(mbuf_v, m_spmem.at[pl.ds(n0 * H, 64 * H)])
    return 0
  lax.fori_loop(0, NPT // 64, sub, 0)
  plsc.subcore_barrier()
  pltpu.sync_copy(m_spmem, m_vmem)

  def issue(base, dstv, lgv, sem):
    return [
        pltpu.async_copy(dst_h.at[pl.ds(base, S)], dstv, sem),
        pltpu.async_copy(logits_h.at[pl.ds(base * H, S * H)], lgv, sem),
    ]

  def compute(base, dstv, lgv, evv, ehs, ihs, sem):
    for g in range(S // 16):
      ridx = iota + g * 16
      dsv = dstv[pl.ds(g * 16, 16)]
      for h in range(H):
        lgh = plsc.load_gather(lgv, [ridx * H + h])
        mv = plsc.load_gather(m_vmem, [dsv * H + h])
        ev = jnp.exp(lgh - mv)
        plsc.store_scatter(evv, [ridx * H + h], ev)
        ehs[h][pl.ds(g * 16, 16)] = ev
        ihs[h][pl.ds(g * 16, 16)] = dsv * H + h
    pltpu.sync_copy(evv, expv_h.at[pl.ds(base * H, S * H)])
    for h in range(H):
      pltpu.sync_copy(ehs[h], den_acc.at[ihs[h]], add=True)
    return []

  def pairD(j, _):
    baseA = w * EPW + (2 * j) * S
    baseB = baseA + S
    cpsA = issue(baseA, dst_v, lg_v, sem1)
    cpsB = issue(baseB, dst2_v, lg2_v, sem2)
    for cp in cpsA:
      cp.wait()
    compute(baseA, dst_v, lg_v, ev_v, eh, ih, sem3)
    for cp in cpsB:
      cp.wait()
    compute(baseB, dst2_v, lg2_v, ev2_v, eh2, ih2, sem1)
    return 0
  lax.fori_loop(0, NPAIR, pairD, 0)

  baseZ = w * EPW + (NCHUNK - 1) * S
  cpsZ = issue(baseZ, dst_v, lg_v, sem1)
  for cp in cpsZ:
    cp.wait()
  compute(baseZ, dst_v, lg_v, ev_v, eh, ih, sem3)

  plsc.subcore_barrier()
  pltpu.sync_copy(den_acc.at[pl.ds(s * NPT * H, NPT * H)], db_v)
  pltpu.sync_copy(db_v, den_h.at[c, pl.ds(s * NPT * H, NPT * H)])


def _e_body(src_h, dst_h, xl_h, expv_h, invd_h,
            outp_h,
            src0_v, dst0_v, src1_v, dst1_v, xl0_v, xl1_v, ev0_v, ev1_v,
            msg_v, msg2_v,
            iv00_v, iv01_v, iv02_v, iv03_v, iv10_v, iv11_v, iv12_v, iv13_v,
            ih00_v, ih01_v, ih02_v, ih03_v, ih10_v, ih11_v, ih12_v, ih13_v,
            out_acc, semA, semB, semC):
  c = lax.axis_index("c")
  s = lax.axis_index("s")
  w = c * NS + s
  iota = lax.iota(I32, 16)
  z16 = jnp.zeros((16,), F32)
  iv = [[iv00_v, iv01_v, iv02_v, iv03_v], [iv10_v, iv11_v, iv12_v, iv13_v]]
  ihb = [[ih00_v, ih01_v, ih02_v, ih03_v], [ih10_v, ih11_v, ih12_v, ih13_v]]

  def fill_zb(i, _):
    msg_v[i // 8, pl.ds((i % 8) * 16, 16)] = z16
    return 0
  lax.fori_loop(0, S * 8, fill_zb, 0)

  def zcp(k, _):
    pltpu.sync_copy(msg_v, out_acc.at[pl.ds(s * NPT + k * S, S)])
    return 0
  lax.fori_loop(0, NPT // S, zcp, 0)
  plsc.subcore_barrier()

  def issue(base, srcv, dstv, xlv, evv, ivs, ihs, sem):
    pltpu.sync_copy(src_h.at[pl.ds(base, S)], srcv)
    pltpu.sync_copy(dst_h.at[pl.ds(base, S)], dstv)
    cps = [
        pltpu.async_copy(xl_h.at[srcv], xlv, sem),
        pltpu.async_copy(expv_h.at[pl.ds(base * H, S * H)], evv, sem),
    ]
    for g in range(S // 16):
      dv = dstv[pl.ds(g * 16, 16)]
      for h in range(H):
        ihs[h][pl.ds(g * 16, 16)] = dv * H + h
    for h in range(H):
      cps.append(pltpu.async_copy(invd_h.at[ihs[h]], ivs[h], sem))
    return cps

  def compute(dstv, xlv, evv, ivs, msgv, sem):
    for g in range(S // 16):
      ridx = iota + g * 16
      alphas = []
      for h in range(H):
        ev = plsc.load_gather(evv, [ridx * H + h])
        alphas.append(ev * ivs[h][pl.ds(g * 16, 16)])
      def mbody(cc):
        dcol = jnp.bitwise_and(iota + cc, CC - 1)
        for h in range(H):
          ch = dcol + h * CC
          xv = plsc.load_gather(xlv, [ridx, ch])
          plsc.store_scatter(msgv, [ridx, ch], xv * alphas[h])
      plsc.parallel_loop(0, CC, 1, unroll=4)(mbody)
    pltpu.sync_copy(msgv, out_acc.at[dstv], add=True)

  def pair(j, _):
    baseA = w * EPW + (2 * j) * S
    baseB = baseA + S
    cpsA = issue(baseA, src0_v, dst0_v, xl0_v, ev0_v, iv[0], ihb[0], semA)
    cpsB = issue(baseB, src1_v, dst1_v, xl1_v, ev1_v, iv[1], ihb[1], semB)
    for cp in cpsA:
      cp.wait()
    compute(dst0_v, xl0_v, ev0_v, iv[0], msg_v, semC)
    for cp in cpsB:
      cp.wait()
    compute(dst1_v, xl1_v, ev1_v, iv[1], msg2_v, semC)
    return 0
  lax.fori_loop(0, NPAIR, pair, 0)

  baseZ = w * EPW + (NCHUNK - 1) * S
  cpsZ = issue(baseZ, src0_v, dst0_v, xl0_v, ev0_v, iv[0], ihb[0], semA)
  for cp in cpsZ:
    cp.wait()
  compute(dst0_v, xl0_v, ev0_v, iv[0], msg_v, semC)

  plsc.subcore_barrier()
  def bounce(k, _):
    r0 = s * NPT + k * S
    pltpu.sync_copy(out_acc.at[pl.ds(r0, S)], msg_v)
    pltpu.sync_copy(msg_v, outp_h.at[c, pl.ds(r0, S)])
    return 0
  lax.fori_loop(0, NPT // S, bounce, 0)


def kernel(x, edge_index, edge_attr, W_ee, b_ee, W_l, b_l, W_r, b_r, W_e,
           att, bias_conv, W_lin, b_lin):
  assert x.shape == (N, D) and edge_attr.shape == (E, D)
  assert att.shape == (H, CC)
  src = edge_index[0]
  dst = edge_index[1]
  attf = att.reshape(-1)
  xpad = jnp.pad(x, ((0, NN - N), (0, 0)))

  mesh = plsc.VectorSubcoreMesh(core_axis_name="c", subcore_axis_name="s",
                                num_cores=NC, num_subcores=NS)

  sega, degp = pl.kernel(
      _b0_body,
      out_type=[
          jax.ShapeDtypeStruct((NC, NN, D), F32),
          jax.ShapeDtypeStruct((NC, NN), F32),
      ],
      mesh=mesh,
      scratch_types=[
          pltpu.VMEM((S,), I32),
          pltpu.VMEM((S,), I32),
          pltpu.VMEM((S, D), F32),
          pltpu.VMEM((S, D), F32),
          pltpu.VMEM((S,), F32),
          pltpu.VMEM((NPT,), F32),
          pltpu.VMEM_SHARED((NN, D), F32),
          pltpu.VMEM_SHARED((NN,), F32),
          pltpu.SemaphoreType.DMA,
          pltpu.SemaphoreType.DMA,
      ],
      **_SCPARAMS,
  )(dst, edge_attr)

  blk = 1024
  xl, xr = pl.pallas_call(
      _a1_body,
      grid=(NN // blk,),
      in_specs=[
          pl.BlockSpec((blk, D), lambda i: (i, 0)),
          pl.BlockSpec((D, D), lambda i: (0, 0)),
          pl.BlockSpec((1, D), lambda i: (0, 0)),
          pl.BlockSpec((D, D), lambda i: (0, 0)),
          pl.BlockSpec((1, D), lambda i: (0, 0)),
      ],
      out_specs=[
          pl.BlockSpec((blk, D), lambda i: (i, 0)),
          pl.BlockSpec((blk, D), lambda i: (i, 0)),
      ],
      out_shape=[
          jax.ShapeDtypeStruct((NN, D), F32),
          jax.ShapeDtypeStruct((NN, D), F32),
      ],
  )(xpad, W_l, b_l.reshape(1, D), W_r, b_r.reshape(1, D))

  eblk = 2560
  ee = pl.pallas_call(
      _a2_body,
      grid=(E // eblk,),
      in_specs=[
          pl.BlockSpec((eblk, D), lambda i: (i, 0)),
          pl.BlockSpec((D, D), lambda i: (0, 0)),
          pl.BlockSpec((1, D), lambda i: (0, 0)),
          pl.BlockSpec((D, D), lambda i: (0, 0)),
      ],
      out_specs=pl.BlockSpec((eblk, D), lambda i: (i, 0)),
      out_shape=jax.ShapeDtypeStruct((E, D), F32),
  )(edge_attr, W_ee, b_ee.reshape(1, D), W_e)

  logits = pl.kernel(
      _b1_body,
      out_type=jax.ShapeDtypeStruct((E * H,), F32),
      mesh=mesh,
      scratch_types=[
          pltpu.VMEM((S,), I32),
          pltpu.VMEM((S,), I32),
          pltpu.VMEM((S,), I32),
          pltpu.VMEM((S,), I32),
          pltpu.VMEM((S, D), F32),
          pltpu.VMEM((S, D), F32),
          pltpu.VMEM((S, D), F32),
          pltpu.VMEM((S, D), F32),
          pltpu.VMEM((S, D), F32),
          pltpu.VMEM((S, D), F32),
          pltpu.VMEM((S * H,), F32),
          pltpu.VMEM((S * H,), F32),
          pltpu.VMEM((D,), F32),
          pltpu.SemaphoreType.DMA,
          pltpu.SemaphoreType.DMA,
          pltpu.SemaphoreType.DMA,
      ],
      **_SCPARAMS,
  )(src, dst, xl, xr, ee, attf)

  loopE = pl.pallas_call(
      _c1_body,
      grid=(NN // blk,),
      in_specs=[
          pl.BlockSpec((blk, D), lambda i: (i, 0)),
          pl.BlockSpec((blk, D), lambda i: (i, 0)),
          pl.BlockSpec((blk, 1), lambda i: (i, 0)),
          pl.BlockSpec((blk, 1), lambda i: (i, 0)),
          pl.BlockSpec((D, D), lambda i: (0, 0)),
          pl.BlockSpec((1, D), lambda i: (0, 0)),
          pl.BlockSpec((D, D), lambda i: (0, 0)),
      ],
      out_specs=pl.BlockSpec((blk, D), lambda i: (i, 0)),
      out_shape=jax.ShapeDtypeStruct((NN, D), F32),
  )(sega[0], sega[1], degp[0].reshape(NN, 1), degp[1].reshape(NN, 1),
    W_ee, b_ee.reshape(1, D), W_e)

  expv, denp = pl.kernel(
      _d_body,
      out_type=[
          jax.ShapeDtypeStruct((E * H,), F32),
          jax.ShapeDtypeStruct((NC, NN * H), F32),
      ],
      mesh=mesh,
      scratch_types=[
          pltpu.VMEM((S,), I32),
          pltpu.VMEM((S * H,), F32),
          pltpu.VMEM((S * H,), F32),
          pltpu.VMEM((S,), I32),
          pltpu.VMEM((S * H,), F32),
          pltpu.VMEM((S * H,), F32),
          pltpu.VMEM((64, D), F32),
          pltpu.VMEM((64, D), F32),
          pltpu.VMEM((64, D), F32),
          pltpu.VMEM((64 * H,), F32),
          pltpu.VMEM((NPT * H,), F32),
          pltpu.VMEM((D,), F32),
          pltpu.VMEM((NN * H,), F32),
          pltpu.VMEM((S,), F32),
          pltpu.VMEM((S,), F32),
          pltpu.VMEM((S,), F32),
          pltpu.VMEM((S,), F32),
          pltpu.VMEM((S,), I32),
          pltpu.VMEM((S,), I32),
          pltpu.VMEM((S,), I32),
          pltpu.VMEM((S,), I32),
          pltpu.VMEM((S,), F32),
          pltpu.VMEM((S,), F32),
          pltpu.VMEM((S,), F32),
          pltpu.VMEM((S,), F32),
          pltpu.VMEM((S,), I32),
          pltpu.VMEM((S,), I32),
          pltpu.VMEM((S,), I32),
          pltpu.VMEM((S,), I32),
          pltpu.VMEM_SHARED((NN * H,), F32),
          pltpu.VMEM_SHARED((NN * H,), F32),
          pltpu.SemaphoreType.DMA,
          pltpu.SemaphoreType.DMA,
          pltpu.SemaphoreType.DMA,
      ],
      **_SCPARAMS,
  )(dst, xl, xr, loopE, attf, logits)

  invd = pl.pallas_call(
      _c2_body,
      grid=(NN // blk,),
      in_specs=[
          pl.BlockSpec((blk, H), lambda i: (i, 0)),
          pl.BlockSpec((blk, H), lambda i: (i, 0)),
      ],
      out_specs=pl.BlockSpec((blk, H), lambda i: (i, 0)),
      out_shape=jax.ShapeDtypeStruct((NN, H), F32),
  )(denp[0].reshape(NN, H), denp[1].reshape(NN, H))

  outp = pl.kernel(
      _e_body,
      out_type=jax.ShapeDtypeStruct((NC, NN, D), F32),
      mesh=mesh,
      scratch_types=[
          pltpu.VMEM((S,), I32),
          pltpu.VMEM((S,), I32),
          pltpu.VMEM((S,), I32),
          pltpu.VMEM((S,), I32),
          pltpu.VMEM((S, D), F32),
          pltpu.VMEM((S, D), F32),
          pltpu.VMEM((S * H,), F32),
          pltpu.VMEM((S * H,), F32),
          pltpu.VMEM((S, D), F32),
          pltpu.VMEM((S, D), F32),
          pltpu.VMEM((S,), F32),
          pltpu.VMEM((S,), F32),
          pltpu.VMEM((S,), F32),
          pltpu.VMEM((S,), F32),
          pltpu.VMEM((S,), F32),
          pltpu.VMEM((S,), F32),
          pltpu.VMEM((S,), F32),
          pltpu.VMEM((S,), F32),
          pltpu.VMEM((S,), I32),
          pltpu.VMEM((S,), I32),
          pltpu.VMEM((S,), I32),
          pltpu.VMEM((S,), I32),
          pltpu.VMEM((S,), I32),
          pltpu.VMEM((S,), I32),
          pltpu.VMEM((S,), I32),
          pltpu.VMEM((S,), I32),
          pltpu.VMEM_SHARED((NN, D), F32),
          pltpu.SemaphoreType.DMA,
          pltpu.SemaphoreType.DMA,
          pltpu.SemaphoreType.DMA,
      ],
      **_SCPARAMS,
  )(src, dst, xl, expv, invd.reshape(-1))

  fblk = 1000
  out = pl.pallas_call(
      _f_body,
      grid=(N // fblk,),
      in_specs=[
          pl.BlockSpec((fblk, D), lambda i: (i, 0)),
          pl.BlockSpec((fblk, D), lambda i: (i, 0)),
          pl.BlockSpec((fblk, H), lambda i: (i, 0)),
          pl.BlockSpec((fblk, D), lambda i: (i, 0)),
          pl.BlockSpec((1, D), lambda i: (0, 0)),
          pl.BlockSpec((D, D), lambda i: (0, 0)),
          pl.BlockSpec((1, D), lambda i: (0, 0)),
      ],
      out_specs=pl.BlockSpec((fblk, D), lambda i: (i, 0)),
      out_shape=jax.ShapeDtypeStruct((N, D), F32),
  )(outp[0], outp[1], invd, xl, bias_conv.reshape(1, D),
    W_lin, b_lin.reshape(1, D))
  return out


# speedup vs baseline: 1.0051x; 1.0051x over previous
import jax
import jax.numpy as jnp
from jax import lax
from jax.experimental import pallas as pl
from jax.experimental.pallas import tpu as pltpu
from jax.experimental.pallas import tpu_sc as plsc

F32 = jnp.float32
I32 = jnp.int32

N, E, D, H, CC = 10000, 320000, 128, 4, 32
NN = 10240
NC, NS, NW = 2, 16, 32
EPW = E // NW
S = 80
NCHUNK = EPW // S
NPAIR = (NCHUNK - 1) // 2
NPT = NN // NS

_SCPARAMS = dict(compiler_params=pltpu.CompilerParams(needs_layout_passes=False))


def _splat(v):
  return jnp.broadcast_to(v, (16,)).astype(I32)


def _a1_body(x_ref, wl_ref, bl_ref, wr_ref, br_ref, xl_ref, xr_ref):
  xb = x_ref[...]
  hp = jax.lax.Precision.HIGHEST
  xl_ref[...] = jnp.dot(xb, wl_ref[...], preferred_element_type=F32,
                        precision=hp) + bl_ref[...]
  xr_ref[...] = jnp.dot(xb, wr_ref[...], preferred_element_type=F32,
                        precision=hp) + br_ref[...]


def _a2_body(ea_ref, wee_ref, bee_ref, we_ref, out_ref):
  hp = jax.lax.Precision.HIGHEST
  t = jnp.dot(ea_ref[...], wee_ref[...], preferred_element_type=F32,
              precision=hp) + bee_ref[...]
  out_ref[...] = jnp.dot(t, we_ref[...], preferred_element_type=F32,
                         precision=hp)


def _c1_body(sa0_ref, sa1_ref, d0_ref, d1_ref, wee_ref, bee_ref, we_ref,
             le_ref):
  hp = jax.lax.Precision.HIGHEST
  dsum = d0_ref[...] + d1_ref[...]
  inv = 1.0 / jnp.maximum(dsum, 1.0)
  mean = (sa0_ref[...] + sa1_ref[...]) * inv
  t = jnp.dot(mean, wee_ref[...], preferred_element_type=F32,
              precision=hp) + bee_ref[...]
  t = jnp.where(dsum > 0.0, t, 0.0)
  le_ref[...] = jnp.dot(t, we_ref[...], preferred_element_type=F32,
                        precision=hp)


def _c2_body(d0_ref, d1_ref, inv_ref):
  d = d0_ref[...] + d1_ref[...] + 1.0
  inv_ref[...] = 1.0 / (d + 1e-16)


def _f_body(o0_ref, o1_ref, inv_ref, xl_ref, bias_ref, wlin_ref,
            blin_ref, out_ref):
  blk = o0_ref.shape[0]
  inv = inv_ref[...]
  cols = [jnp.broadcast_to(inv[:, h:h + 1], (blk, CC)) for h in range(H)]
  al = jnp.concatenate(cols, axis=1)
  acc = o0_ref[...] + o1_ref[...] + xl_ref[...] * al + bias_ref[...]
  acc = jnp.maximum(acc, 0.0)
  out_ref[...] = jnp.dot(acc, wlin_ref[...], preferred_element_type=F32,
                         precision=jax.lax.Precision.HIGHEST) + blin_ref[...]


def _b0_body(dst_h, ea_h, sega_h, deg_h,
             dst0_v, dst1_v, ea0_v, ea1_v, ones_v, dbuf_v,
             sega_acc, deg_acc, semA, semB):
  c = lax.axis_index("c")
  s = lax.axis_index("s")
  w = c * NS + s
  z16 = jnp.zeros((16,), F32)
  o16 = jnp.ones((16,), F32)

  def fill_ones(i, _):
    ones_v[pl.ds(i * 16, 16)] = o16
    return 0
  lax.fori_loop(0, S // 16, fill_ones, 0)

  def fill_zb(i, _):
    ea0_v[i // 8, pl.ds((i % 8) * 16, 16)] = z16
    return 0
  lax.fori_loop(0, S * 8, fill_zb, 0)

  def fill_db(i, _):
    dbuf_v[pl.ds(i * 16, 16)] = z16
    return 0
  lax.fori_loop(0, NPT // 16, fill_db, 0)

  def zcp(k, _):
    pltpu.sync_copy(ea0_v, sega_acc.at[pl.ds(s * NPT + k * S, S)])
    return 0
  lax.fori_loop(0, NPT // S, zcp, 0)
  pltpu.sync_copy(dbuf_v, deg_acc.at[pl.ds(s * NPT, NPT)])
  plsc.subcore_barrier()

  def pair(j, _):
    baseA = w * EPW + (2 * j) * S
    baseB = baseA + S
    cpA = pltpu.async_copy(ea_h.at[pl.ds(baseA, S)], ea0_v, semA)
    cpB = pltpu.async_copy(ea_h.at[pl.ds(baseB, S)], ea1_v, semB)
    pltpu.sync_copy(dst_h.at[pl.ds(baseA, S)], dst0_v)
    pltpu.sync_copy(dst_h.at[pl.ds(baseB, S)], dst1_v)
    cpA.wait()
    pltpu.sync_copy(ea0_v, sega_acc.at[dst0_v], add=True)
    pltpu.sync_copy(ones_v, deg_acc.at[dst0_v], add=True)
    cpB.wait()
    pltpu.sync_copy(ea1_v, sega_acc.at[dst1_v], add=True)
    pltpu.sync_copy(ones_v, deg_acc.at[dst1_v], add=True)
    return 0
  lax.fori_loop(0, NPAIR, pair, 0)

  baseZ = w * EPW + (NCHUNK - 1) * S
  cpZ = pltpu.async_copy(ea_h.at[pl.ds(baseZ, S)], ea0_v, semA)
  pltpu.sync_copy(dst_h.at[pl.ds(baseZ, S)], dst0_v)
  cpZ.wait()
  pltpu.sync_copy(ea0_v, sega_acc.at[dst0_v], add=True)
  pltpu.sync_copy(ones_v, deg_acc.at[dst0_v], add=True)

  plsc.subcore_barrier()
  def bounce(k, _):
    r0 = s * NPT + k * S
    pltpu.sync_copy(sega_acc.at[pl.ds(r0, S)], ea0_v)
    pltpu.sync_copy(ea0_v, sega_h.at[c, pl.ds(r0, S)])
    return 0
  lax.fori_loop(0, NPT // S, bounce, 0)
  pltpu.sync_copy(deg_acc.at[pl.ds(s * NPT, NPT)], dbuf_v)
  pltpu.sync_copy(dbuf_v, deg_h.at[c, pl.ds(s * NPT, NPT)])


def _bd_body(src_h, dst_h, xl_h, xr_h, ee_h, le_h, att_h,
             expv_h, den_h,
             src0_v, dst0_v, src1_v, dst1_v,
             xl0_v, xr0_v, ee0_v, xl1_v, xr1_v, ee1_v,
             lg_v, lg2_v, ev_v, ev2_v, db_v, att_v, m_vmem,
             eh0_v, eh1_v, eh2_v, eh3_v, ih0_v, ih1_v, ih2_v, ih3_v,
             eh4_v, eh5_v, eh6_v, eh7_v, ih4_v, ih5_v, ih6_v, ih7_v,
             m_spmem, den_acc, semA, semB, semC):
  c = lax.axis_index("c")
  s = lax.axis_index("s")
  w = c * NS + s
  iota = lax.iota(I32, 16)
  z16 = jnp.zeros((16,), F32)
  eh = [eh0_v, eh1_v, eh2_v, eh3_v]
  ih = [ih0_v, ih1_v, ih2_v, ih3_v]
  eh2s = [eh4_v, eh5_v, eh6_v, eh7_v]
  ih2s = [ih4_v, ih5_v, ih6_v, ih7_v]

  pltpu.sync_copy(att_h, att_v)

  def fill_db(i, _):
    db_v[pl.ds(i * 16, 16)] = z16
    return 0
  lax.fori_loop(0, (NPT * H) // 16, fill_db, 0)
  pltpu.sync_copy(db_v, den_acc.at[pl.ds(s * NPT * H, NPT * H)])

  def logits_into(xlv, xrv, eev, lgv):
    for g in range(S // 16):
      ridx = iota + g * 16
      def cbody(cc, carry):
        dcol = jnp.bitwise_and(iota + cc, CC - 1)
        accs = []
        for h in range(H):
          ch = dcol + h * CC
          a = plsc.load_gather(xlv, [ridx, ch])
          b = plsc.load_gather(xrv, [ridx, ch])
          e = plsc.load_gather(eev, [ridx, ch])
          av = plsc.load_gather(att_v, [ch])
          f = a + b + e
          f = jnp.maximum(f, 0.2 * f)
          accs.append(carry[h] + f * av)
        return tuple(accs)
      accs = plsc.parallel_loop(0, CC, 1, unroll=4,
                                carry=(z16, z16, z16, z16))(cbody)
      for h in range(H):
        plsc.store_scatter(lgv, [ridx * H + h], accs[h])

  def sub(k, _):
    n0 = s * NPT + k * S
    cp1 = pltpu.async_copy(xl_h.at[pl.ds(n0, S)], xl0_v, semA)
    cp2 = pltpu.async_copy(xr_h.at[pl.ds(n0, S)], xr0_v, semB)
    cp3 = pltpu.async_copy(le_h.at[pl.ds(n0, S)], ee0_v, semC)
    cp1.wait()
    cp2.wait()
    cp3.wait()
    logits_into(xl0_v, xr0_v, ee0_v, lg_v)
    pltpu.sync_copy(lg_v, m_spmem.at[pl.ds(n0 * H, S * H)])
    return 0
  lax.fori_loop(0, NPT // S, sub, 0)
  plsc.subcore_barrier()
  pltpu.sync_copy(m_spmem, m_vmem)

  def issue(base, srcv, dstv, xlv, xrv, eev, sem):
    pltpu.sync_copy(src_h.at[pl.ds(base, S)], srcv)
    pltpu.sync_copy(dst_h.at[pl.ds(base, S)], dstv)
    return [
        pltpu.async_copy(xl_h.at[srcv], xlv, sem),
        pltpu.async_copy(xr_h.at[dstv], xrv, sem),
        pltpu.async_copy(ee_h.at[pl.ds(base, S)], eev, sem),
    ]

  def compute(base, dstv, xlv, xrv, eev, lgv, evv, ehs, ihs):
    logits_into(xlv, xrv, eev, lgv)
    for g in range(S // 16):
      ridx = iota + g * 16
      dsv = dstv[pl.ds(g * 16, 16)]
      for h in range(H):
        lgh = plsc.load_gather(lgv, [ridx * H + h])
        mv = plsc.load_gather(m_vmem, [dsv * H + h])
        ev = jnp.exp(lgh - mv)
        plsc.store_scatter(evv, [ridx * H + h], ev)
        ehs[h][pl.ds(g * 16, 16)] = ev
        ihs[h][pl.ds(g * 16, 16)] = dsv * H + h
    pltpu.sync_copy(evv, expv_h.at[pl.ds(base * H, S * H)])
    for h in range(H):
      pltpu.sync_copy(ehs[h], den_acc.at[ihs[h]], add=True)

  def pair(j, _):
    baseA = w * EPW + (2 * j) * S
    baseB = baseA + S
    cpsA = issue(baseA, src0_v, dst0_v, xl0_v, xr0_v, ee0_v, semA)
    cpsB = issue(baseB, src1_v, dst1_v, xl1_v, xr1_v, ee1_v, semB)
    for cp in cpsA:
      cp.wait()
    compute(baseA, dst0_v, xl0_v, xr0_v, ee0_v, lg_v, ev_v, eh, ih)
    for cp in cpsB:
      cp.wait()
    compute(baseB, dst1_v, xl1_v, xr1_v, ee1_v, lg2_v, ev2_v, eh2s, ih2s)
    return 0
  lax.fori_loop(0, NPAIR, pair, 0)

  baseZ = w * EPW + (NCHUNK - 1) * S
  cpsZ = issue(baseZ, src0_v, dst0_v, xl0_v, xr0_v, ee0_v, semA)
  for cp in cpsZ:
    cp.wait()
  compute(baseZ, dst0_v, xl0_v, xr0_v, ee0_v, lg_v, ev_v, eh, ih)

  plsc.subcore_barrier()
  pltpu.sync_copy(den_acc.at[pl.ds(s * NPT * H, NPT * H)], db_v)
  pltpu.sync_copy(db_v, den_h.at[c, pl.ds(s * NPT * H, NPT * H)])


def _e_body(src_h, dst_h, xl_h, expv_h, invd_h,
            outp_h,
            src0_v, dst0_v, src1_v, dst1_v, xl0_v, xl1_v, ev0_v, ev1_v,
            msg_v, msg2_v,
            iv00_v, iv01_v, iv02_v, iv03_v, iv10_v, iv11_v, iv12_v, iv13_v,
            ih00_v, ih01_v, ih02_v, ih03_v, ih10_v, ih11_v, ih12_v, ih13_v,
            out_acc, semA, semB, semC):
  c = lax.axis_index("c")
  s = lax.axis_index("s")
  w = c * NS + s
  iota = lax.iota(I32, 16)
  z16 = jnp.zeros((16,), F32)
  iv = [[iv00_v, iv01_v, iv02_v, iv03_v], [iv10_v, iv11_v, iv12_v, iv13_v]]
  ihb = [[ih00_v, ih01_v, ih02_v, ih03_v], [ih10_v, ih11_v, ih12_v, ih13_v]]

  def fill_zb(i, _):
    msg_v[i // 8, pl.ds((i % 8) * 16, 16)] = z16
    return 0
  lax.fori_loop(0, S * 8, fill_zb, 0)

  def zcp(k, _):
    pltpu.sync_copy(msg_v, out_acc.at[pl.ds(s * NPT + k * S, S)])
    return 0
  lax.fori_loop(0, NPT // S, zcp, 0)
  plsc.subcore_barrier()

  def issue(base, srcv, dstv, xlv, evv, ivs, ihs, sem):
    pltpu.sync_copy(src_h.at[pl.ds(base, S)], srcv)
    pltpu.sync_copy(dst_h.at[pl.ds(base, S)], dstv)
    cps = [
        pltpu.async_copy(xl_h.at[srcv], xlv, sem),
        pltpu.async_copy(expv_h.at[pl.ds(base * H, S * H)], evv, sem),
    ]
    for g in range(S // 16):
      dv = dstv[pl.ds(g * 16, 16)]
      for h in range(H):
        ihs[h][pl.ds(g * 16, 16)] = dv * H + h
    for h in range(H):
      cps.append(pltpu.async_copy(invd_h.at[ihs[h]], ivs[h], sem))
    return cps

  def compute(dstv, xlv, evv, ivs, msgv, sem):
    for g in range(S // 16):
      ridx = iota + g * 16
      alphas = []
      for h in range(H):
        ev = plsc.load_gather(evv, [ridx * H + h])
        alphas.append(ev * ivs[h][pl.ds(g * 16, 16)])
      def mbody(cc):
        dcol = jnp.bitwise_and(iota + cc, CC - 1)
        for h in range(H):
          ch = dcol + h * CC
          xv = plsc.load_gather(xlv, [ridx, ch])
          plsc.store_scatter(msgv, [ridx, ch], xv * alphas[h])
      plsc.parallel_loop(0, CC, 1, unroll=4)(mbody)
    pltpu.sync_copy(msgv, out_acc.at[dstv], add=True)

  def pair(j, _):
    baseA = w * EPW + (2 * j) * S
    baseB = baseA + S
    cpsA = issue(baseA, src0_v, dst0_v, xl0_v, ev0_v, iv[0], ihb[0], semA)
    cpsB = issue(baseB, src1_v, dst1_v, xl1_v, ev1_v, iv[1], ihb[1], semB)
    for cp in cpsA:
      cp.wait()
    compute(dst0_v, xl0_v, ev0_v, iv[0], msg_v, semC)
    for cp in cpsB:
      cp.wait()
    compute(dst1_v, xl1_v, ev1_v, iv[1], msg2_v, semC)
    return 0
  lax.fori_loop(0, NPAIR, pair, 0)

  baseZ = w * EPW + (NCHUNK - 1) * S
  cpsZ = issue(baseZ, src0_v, dst0_v, xl0_v, ev0_v, iv[0], ihb[0], semA)
  for cp in cpsZ:
    cp.wait()
  compute(dst0_v, xl0_v, ev0_v, iv[0], msg_v, semC)

  plsc.subcore_barrier()
  def bounce(k, _):
    r0 = s * NPT + k * S
    pltpu.sync_copy(out_acc.at[pl.ds(r0, S)], msg_v)
    pltpu.sync_copy(msg_v, outp_h.at[c, pl.ds(r0, S)])
    return 0
  lax.fori_loop(0, NPT // S, bounce, 0)


def kernel(x, edge_index, edge_attr, W_ee, b_ee, W_l, b_l, W_r, b_r, W_e,
           att, bias_conv, W_lin, b_lin):
  assert x.shape == (N, D) and edge_attr.shape == (E, D)
  assert att.shape == (H, CC)
  src = edge_index[0]
  dst = edge_index[1]
  attf = att.reshape(-1)
  xpad = jnp.pad(x, ((0, NN - N), (0, 0)))

  mesh = plsc.VectorSubcoreMesh(core_axis_name="c", subcore_axis_name="s",
                                num_cores=NC, num_subcores=NS)

  sega, degp = pl.kernel(
      _b0_body,
      out_type=[
          jax.ShapeDtypeStruct((NC, NN, D), F32),
          jax.ShapeDtypeStruct((NC, NN), F32),
      ],
      mesh=mesh,
      scratch_types=[
          pltpu.VMEM((S,), I32),
          pltpu.VMEM((S,), I32),
          pltpu.VMEM((S, D), F32),
          pltpu.VMEM((S, D), F32),
          pltpu.VMEM((S,), F32),
          pltpu.VMEM((NPT,), F32),
          pltpu.VMEM_SHARED((NN, D), F32),
          pltpu.VMEM_SHARED((NN,), F32),
          pltpu.SemaphoreType.DMA,
          pltpu.SemaphoreType.DMA,
      ],
      **_SCPARAMS,
  )(dst, edge_attr)

  blk = 1024
  xl, xr = pl.pallas_call(
      _a1_body,
      grid=(NN // blk,),
      in_specs=[
          pl.BlockSpec((blk, D), lambda i: (i, 0)),
          pl.BlockSpec((D, D), lambda i: (0, 0)),
          pl.BlockSpec((1, D), lambda i: (0, 0)),
          pl.BlockSpec((D, D), lambda i: (0, 0)),
          pl.BlockSpec((1, D), lambda i: (0, 0)),
      ],
      out_specs=[
          pl.BlockSpec((blk, D), lambda i: (i, 0)),
          pl.BlockSpec((blk, D), lambda i: (i, 0)),
      ],
      out_shape=[
          jax.ShapeDtypeStruct((NN, D), F32),
          jax.ShapeDtypeStruct((NN, D), F32),
      ],
  )(xpad, W_l, b_l.reshape(1, D), W_r, b_r.reshape(1, D))

  eblk = 2560
  ee = pl.pallas_call(
      _a2_body,
      grid=(E // eblk,),
      in_specs=[
          pl.BlockSpec((eblk, D), lambda i: (i, 0)),
          pl.BlockSpec((D, D), lambda i: (0, 0)),
          pl.BlockSpec((1, D), lambda i: (0, 0)),
          pl.BlockSpec((D, D), lambda i: (0, 0)),
      ],
      out_specs=pl.BlockSpec((eblk, D), lambda i: (i, 0)),
      out_shape=jax.ShapeDtypeStruct((E, D), F32),
  )(edge_attr, W_ee, b_ee.reshape(1, D), W_e)

  loopE = pl.pallas_call(
      _c1_body,
      grid=(NN // blk,),
      in_specs=[
          pl.BlockSpec((blk, D), lambda i: (i, 0)),
          pl.BlockSpec((blk, D), lambda i: (i, 0)),
          pl.BlockSpec((blk, 1), lambda i: (i, 0)),
          pl.BlockSpec((blk, 1), lambda i: (i, 0)),
          pl.BlockSpec((D, D), lambda i: (0, 0)),
          pl.BlockSpec((1, D), lambda i: (0, 0)),
          pl.BlockSpec((D, D), lambda i: (0, 0)),
      ],
      out_specs=pl.BlockSpec((blk, D), lambda i: (i, 0)),
      out_shape=jax.ShapeDtypeStruct((NN, D), F32),
  )(sega[0], sega[1], degp[0].reshape(NN, 1), degp[1].reshape(NN, 1),
    W_ee, b_ee.reshape(1, D), W_e)

  expv, denp = pl.kernel(
      _bd_body,
      out_type=[
          jax.ShapeDtypeStruct((E * H,), F32),
          jax.ShapeDtypeStruct((NC, NN * H), F32),
      ],
      mesh=mesh,
      scratch_types=[
          pltpu.VMEM((S,), I32),
          pltpu.VMEM((S,), I32),
          pltpu.VMEM((S,), I32),
          pltpu.VMEM((S,), I32),
          pltpu.VMEM((S, D), F32),
          pltpu.VMEM((S, D), F32),
          pltpu.VMEM((S, D), F32),
          pltpu.VMEM((S, D), F32),
          pltpu.VMEM((S, D), F32),
          pltpu.VMEM((S, D), F32),
          pltpu.VMEM((S * H,), F32),
          pltpu.VMEM((S * H,), F32),
          pltpu.VMEM((S * H,), F32),
          pltpu.VMEM((S * H,), F32),
          pltpu.VMEM((NPT * H,), F32),
          pltpu.VMEM((D,), F32),
          pltpu.VMEM((NN * H,), F32),
          pltpu.VMEM((S,), F32),
          pltpu.VMEM((S,), F32),
          pltpu.VMEM((S,), F32),
          pltpu.VMEM((S,), F32),
          pltpu.VMEM((S,), I32),
          pltpu.VMEM((S,), I32),
          pltpu.VMEM((S,), I32),
          pltpu.VMEM((S,), I32),
          pltpu.VMEM((S,), F32),
          pltpu.VMEM((S,), F32),
          pltpu.VMEM((S,), F32),
          pltpu.VMEM((S,), F32),
          pltpu.VMEM((S,), I32),
          pltpu.VMEM((S,), I32),
          pltpu.VMEM((S,), I32),
          pltpu.VMEM((S,), I32),
          pltpu.VMEM_SHARED((NN * H,), F32),
          pltpu.VMEM_SHARED((NN * H,), F32),
          pltpu.SemaphoreType.DMA,
          pltpu.SemaphoreType.DMA,
          pltpu.SemaphoreType.DMA,
      ],
      **_SCPARAMS,
  )(src, dst, xl, xr, ee, loopE, attf)

  invd = pl.pallas_call(
      _c2_body,
      grid=(NN // blk,),
      in_specs=[
          pl.BlockSpec((blk, H), lambda i: (i, 0)),
          pl.BlockSpec((blk, H), lambda i: (i, 0)),
      ],
      out_specs=pl.BlockSpec((blk, H), lambda i: (i, 0)),
      out_shape=jax.ShapeDtypeStruct((NN, H), F32),
  )(denp[0].reshape(NN, H), denp[1].reshape(NN, H))

  outp = pl.kernel(
      _e_body,
      out_type=jax.ShapeDtypeStruct((NC, NN, D), F32),
      mesh=mesh,
      scratch_types=[
          pltpu.VMEM((S,), I32),
          pltpu.VMEM((S,), I32),
          pltpu.VMEM((S,), I32),
          pltpu.VMEM((S,), I32),
          pltpu.VMEM((S, D), F32),
          pltpu.VMEM((S, D), F32),
          pltpu.VMEM((S * H,), F32),
          pltpu.VMEM((S * H,), F32),
          pltpu.VMEM((S, D), F32),
          pltpu.VMEM((S, D), F32),
          pltpu.VMEM((S,), F32),
          pltpu.VMEM((S,), F32),
          pltpu.VMEM((S,), F32),
          pltpu.VMEM((S,), F32),
          pltpu.VMEM((S,), F32),
          pltpu.VMEM((S,), F32),
          pltpu.VMEM((S,), F32),
          pltpu.VMEM((S,), F32),
          pltpu.VMEM((S,), I32),
          pltpu.VMEM((S,), I32),
          pltpu.VMEM((S,), I32),
          pltpu.VMEM((S,), I32),
          pltpu.VMEM((S,), I32),
          pltpu.VMEM((S,), I32),
          pltpu.VMEM((S,), I32),
          pltpu.VMEM((S,), I32),
          pltpu.VMEM_SHARED((NN, D), F32),
          pltpu.SemaphoreType.DMA,
          pltpu.SemaphoreType.DMA,
          pltpu.SemaphoreType.DMA,
      ],
      **_SCPARAMS,
  )(src, dst, xl, expv, invd.reshape(-1))

  fblk = 1000
  out = pl.pallas_call(
      _f_body,
      grid=(N // fblk,),
      in_specs=[
          pl.BlockSpec((fblk, D), lambda i: (i, 0)),
          pl.BlockSpec((fblk, D), lambda i: (i, 0)),
          pl.BlockSpec((fblk, H), lambda i: (i, 0)),
          pl.BlockSpec((fblk, D), lambda i: (i, 0)),
          pl.BlockSpec((1, D), lambda i: (0, 0)),
          pl.BlockSpec((D, D), lambda i: (0, 0)),
          pl.BlockSpec((1, D), lambda i: (0, 0)),
      ],
      out_specs=pl.BlockSpec((fblk, D), lambda i: (i, 0)),
      out_shape=jax.ShapeDtypeStruct((N, D), F32),
  )(outp[0], outp[1], invd, xl, bias_conv.reshape(1, D),
    W_lin, b_lin.reshape(1, D))
  return out
